# Initial kernel scaffold; baseline (speedup 1.0000x reference)
import jax
import jax.numpy as jnp
from jax.experimental import pallas as pl


def kernel(x, edge_index, batch, W_l, b_l, W_r, W2, b2):
    raise NotImplementedError("write your pallas kernel here")


# speedup vs baseline: 2.9601x; 2.9601x over previous
import jax
import jax.numpy as jnp
from jax import lax
from jax.experimental import pallas as pl
from jax.experimental.pallas import tpu as pltpu
from jax.experimental.pallas import tpu_sc as plsc

N = 10000
E = 320000
D = 128
H = 128
C = 2
G = 64

NC = 2
NS = 16
NW = NC * NS
CHUNK = 128
CHUNKS_PER_W = 80
EP = NW * CHUNKS_PER_W * CHUNK
RP = 10240
ROWS_PER_S = RP // NS

TC_BLK = 1024
TC_GRID = RP // TC_BLK

CNT_BLK = 8000
CNT_GRID = E // CNT_BLK
CNT_HI = RP // 128


def _sc_body(x_hbm, src_hbm, dst_hbm, z128,
             agg_out, src_v, dst_v, rows_v, agg_sp, sem):
    c = lax.axis_index("c")
    s = lax.axis_index("s")
    wid = s * NC + c
    base_r = s * ROWS_PER_S

    def zbody(k, carry):
        pltpu.sync_copy(z128, agg_sp.at[pl.ds(base_r + k * 128, 128)])
        return carry

    lax.fori_loop(0, ROWS_PER_S // 128, zbody, 0)
    plsc.subcore_barrier()

    base_e = wid * (CHUNKS_PER_W * CHUNK)

    def ebody(i, carry):
        off = base_e + i * CHUNK
        pltpu.sync_copy(src_hbm.at[pl.ds(off, CHUNK)], src_v)
        pltpu.sync_copy(dst_hbm.at[pl.ds(off, CHUNK)], dst_v)
        pltpu.async_copy(x_hbm.at[src_v], rows_v, sem).wait()
        pltpu.sync_copy(rows_v, agg_sp.at[dst_v], add=True)
        return carry

    lax.fori_loop(0, CHUNKS_PER_W, ebody, 0)
    plsc.subcore_barrier()

    pltpu.sync_copy(agg_sp.at[pl.ds(base_r, ROWS_PER_S)],
                    agg_out.at[c, pl.ds(base_r, ROWS_PER_S)])


def _sc_aggregate(x, src_p, dst_p):
    mesh = plsc.VectorSubcoreMesh(core_axis_name="c", subcore_axis_name="s")
    z128 = jnp.zeros((128, D), jnp.float32)
    fn = pl.kernel(
        _sc_body,
        out_type=[jax.ShapeDtypeStruct((NC, RP, D), jnp.float32)],
        mesh=mesh,
        scratch_types=[
            pltpu.VMEM((CHUNK,), jnp.int32),
            pltpu.VMEM((CHUNK,), jnp.int32),
            pltpu.VMEM((CHUNK, D), jnp.float32),
            pltpu.VMEM_SHARED((RP, D), jnp.float32),
            pltpu.SemaphoreType.DMA,
        ],
    )
    return fn(x, src_p, dst_p, z128)[0]


def _cnt_body(dst_ref, out_ref, acc):
    i = pl.program_id(0)

    @pl.when(i == 0)
    def _():
        acc[...] = jnp.zeros_like(acc)

    d = dst_ref[...]
    hi = lax.shift_right_logical(d, 7)
    lo = jnp.bitwise_and(d, 127)
    a = (hi == lax.broadcasted_iota(jnp.int32, (CNT_BLK, CNT_HI), 1))
    b = (lo == lax.broadcasted_iota(jnp.int32, (CNT_BLK, 128), 1))
    acc[...] += lax.dot_general(
        a.astype(jnp.bfloat16), b.astype(jnp.bfloat16),
        (((0,), (0,)), ((), ())),
        preferred_element_type=jnp.float32)

    @pl.when(i == CNT_GRID - 1)
    def _():
        out_ref[...] = acc[...]


def _tc_count(dst2d):
    return pl.pallas_call(
        _cnt_body,
        grid=(CNT_GRID,),
        in_specs=[pl.BlockSpec((CNT_BLK, 1), lambda i: (i, 0))],
        out_specs=pl.BlockSpec((CNT_HI, 128), lambda i: (0, 0)),
        out_shape=jax.ShapeDtypeStruct((CNT_HI, 128), jnp.float32),
        scratch_shapes=[pltpu.VMEM((CNT_HI, 128), jnp.float32)],
    )(dst2d)


def _tc_body(agg_ref, cnt_ref, x_ref, b_ref, wl_ref, bl_ref, wr_ref,
             w2_ref, b2_ref, out_ref, pooled):
    i = pl.program_id(0)

    @pl.when(i == 0)
    def _():
        pooled[...] = jnp.zeros_like(pooled)

    a = agg_ref[0] + agg_ref[1]
    cnt = cnt_ref[...]
    mean = a / jnp.maximum(cnt, 1.0)
    h = mean @ wl_ref[...] + bl_ref[...] + x_ref[...] @ wr_ref[...]
    h = jnp.maximum(h, 0.0)
    bcol = b_ref[...]
    parts = []
    for g in range(G):
        hg = jnp.where(bcol == jnp.float32(g), h, 0.0)
        parts.append(jnp.max(hg, axis=0, keepdims=True))
    blockpool = jnp.concatenate(parts, axis=0)
    pooled[...] = jnp.maximum(pooled[...], blockpool)

    @pl.when(i == TC_GRID - 1)
    def _():
        logits = pooled[...] @ w2_ref[...] + b2_ref[...]
        m = jnp.max(logits, axis=-1, keepdims=True)
        lse = jnp.log(jnp.sum(jnp.exp(logits - m), axis=-1, keepdims=True)) + m
        out_ref[...] = logits - lse


def _tc_head(agg2, cnt2d, x_pad, bcol2d, W_l, b_l, W_r, W2, b2):
    return pl.pallas_call(
        _tc_body,
        grid=(TC_GRID,),
        in_specs=[
            pl.BlockSpec((NC, TC_BLK, D), lambda i: (0, i, 0)),
            pl.BlockSpec((TC_BLK, 1), lambda i: (i, 0)),
            pl.BlockSpec((TC_BLK, D), lambda i: (i, 0)),
            pl.BlockSpec((TC_BLK, 1), lambda i: (i, 0)),
            pl.BlockSpec((D, H), lambda i: (0, 0)),
            pl.BlockSpec((1, H), lambda i: (0, 0)),
            pl.BlockSpec((D, H), lambda i: (0, 0)),
            pl.BlockSpec((H, C), lambda i: (0, 0)),
            pl.BlockSpec((1, C), lambda i: (0, 0)),
        ],
        out_specs=pl.BlockSpec((G, C), lambda i: (0, 0)),
        out_shape=jax.ShapeDtypeStruct((G, C), jnp.float32),
        scratch_shapes=[pltpu.VMEM((G, H), jnp.float32)],
    )(agg2, cnt2d, x_pad, bcol2d, W_l, b_l, W_r, W2, b2)


@jax.jit
def kernel(x, edge_index, batch, W_l, b_l, W_r, W2, b2):
    src = edge_index[0].astype(jnp.int32)
    dst = edge_index[1].astype(jnp.int32)
    pad_e = EP - E
    src_p = jnp.concatenate([src, jnp.zeros((pad_e,), jnp.int32)])
    dst_p = jnp.concatenate([dst, jnp.full((pad_e,), N, jnp.int32)])

    agg2 = _sc_aggregate(x, src_p, dst_p)

    cnt_plane = _tc_count(dst.reshape(E, 1))
    cnt2d = cnt_plane.reshape(RP, 1)

    x_pad = jnp.concatenate([x, jnp.zeros((RP - N, D), jnp.float32)])
    bcol = jnp.concatenate([batch.astype(jnp.float32),
                            jnp.full((RP - N,), jnp.float32(G))])
    return _tc_head(agg2, cnt2d, x_pad, bcol[:, None],
                    W_l, b_l.reshape(1, H), W_r, W2, b2.reshape(1, C))


# speedup vs baseline: 3.4096x; 1.1519x over previous
import jax
import jax.numpy as jnp
from jax import lax
from jax.experimental import pallas as pl
from jax.experimental.pallas import tpu as pltpu
from jax.experimental.pallas import tpu_sc as plsc

N = 10000
E = 320000
D = 128
H = 128
C = 2
G = 64

NC = 2
NS = 16
NW = NC * NS
CHUNK = 128
CHUNKS_PER_W = 80
EP = NW * CHUNKS_PER_W * CHUNK
RP = 10240
ROWS_PER_S = RP // NS

TC_BLK = 1024
TC_GRID = RP // TC_BLK

CNT_BLK = 8000
CNT_GRID = E // CNT_BLK
CNT_HI = RP // 128


NBUF = 2


def _sc_body(x_hbm, src_hbm, dst_hbm, z128,
             agg_out, src_v, dst_v, rows_a, rows_b,
             agg_sp, sem_a, sem_b):
    rows = (rows_a, rows_b)
    sems = (sem_a, sem_b)
    c = lax.axis_index("c")
    s = lax.axis_index("s")
    wid = s * NC + c
    base_r = s * ROWS_PER_S

    def zbody(k, carry):
        pltpu.sync_copy(z128, agg_sp.at[pl.ds(base_r + k * 128, 128)])
        return carry

    lax.fori_loop(0, ROWS_PER_S // 128, zbody, 0)
    plsc.subcore_barrier()

    HALF = CHUNKS_PER_W // 2
    for ph in range(2):
        poff = wid * CHUNKS_PER_W + ph * HALF
        pltpu.sync_copy(src_hbm.at[pl.ds(poff, HALF)], src_v)
        pltpu.sync_copy(dst_hbm.at[pl.ds(poff, HALF)], dst_v)
        for b in range(NBUF):
            pltpu.async_copy(x_hbm.at[src_v.at[b]], rows[b], sems[b])

        def pair(j, carry):
            for b in range(NBUF):
                i = j * NBUF + b
                pltpu.make_async_copy(x_hbm.at[src_v.at[i]], rows[b],
                                      sems[b]).wait()
                pltpu.sync_copy(rows[b], agg_sp.at[dst_v.at[i]], add=True)
                nxt = i + NBUF

                @pl.when(nxt < HALF)
                def _():
                    pltpu.async_copy(x_hbm.at[src_v.at[nxt]], rows[b], sems[b])
            return carry

        lax.fori_loop(0, HALF // NBUF, pair, 0)
    plsc.subcore_barrier()

    pltpu.sync_copy(agg_sp.at[pl.ds(base_r, ROWS_PER_S)],
                    agg_out.at[c, pl.ds(base_r, ROWS_PER_S)])


def _sc_aggregate(x, src_p, dst_p):
    mesh = plsc.VectorSubcoreMesh(core_axis_name="c", subcore_axis_name="s")
    z128 = jnp.zeros((128, D), jnp.float32)
    fn = pl.kernel(
        _sc_body,
        out_type=[jax.ShapeDtypeStruct((NC, RP, D), jnp.float32)],
        mesh=mesh,
        scratch_types=[
            pltpu.VMEM((CHUNKS_PER_W // 2, CHUNK), jnp.int32),
            pltpu.VMEM((CHUNKS_PER_W // 2, CHUNK), jnp.int32),
            pltpu.VMEM((CHUNK, D), jnp.float32),
            pltpu.VMEM((CHUNK, D), jnp.float32),
            pltpu.VMEM_SHARED((RP, D), jnp.float32),
            pltpu.SemaphoreType.DMA,
            pltpu.SemaphoreType.DMA,
        ],
    )
    return fn(x, src_p.reshape(EP // CHUNK, CHUNK),
              dst_p.reshape(EP // CHUNK, CHUNK), z128)[0]


def _cnt_body(dst_ref, out_ref, acc):
    i = pl.program_id(0)

    @pl.when(i == 0)
    def _():
        acc[...] = jnp.zeros_like(acc)

    d = dst_ref[...]
    hi = lax.shift_right_logical(d, 7)
    lo = jnp.bitwise_and(d, 127)
    a = (hi == lax.broadcasted_iota(jnp.int32, (CNT_BLK, CNT_HI), 1))
    b = (lo == lax.broadcasted_iota(jnp.int32, (CNT_BLK, 128), 1))
    acc[...] += lax.dot_general(
        a.astype(jnp.bfloat16), b.astype(jnp.bfloat16),
        (((0,), (0,)), ((), ())),
        preferred_element_type=jnp.float32)

    @pl.when(i == CNT_GRID - 1)
    def _():
        out_ref[...] = acc[...]


def _tc_count(dst2d):
    return pl.pallas_call(
        _cnt_body,
        grid=(CNT_GRID,),
        in_specs=[pl.BlockSpec((CNT_BLK, 1), lambda i: (i, 0))],
        out_specs=pl.BlockSpec((CNT_HI, 128), lambda i: (0, 0)),
        out_shape=jax.ShapeDtypeStruct((CNT_HI, 128), jnp.float32),
        scratch_shapes=[pltpu.VMEM((CNT_HI, 128), jnp.float32)],
    )(dst2d)


def _tc_body(agg_ref, cnt_ref, x_ref, b_ref, wl_ref, bl_ref, wr_ref,
             w2_ref, b2_ref, out_ref, pooled):
    i = pl.program_id(0)

    @pl.when(i == 0)
    def _():
        pooled[...] = jnp.zeros_like(pooled)

    a = agg_ref[0] + agg_ref[1]
    cnt = cnt_ref[...]
    mean = a / jnp.maximum(cnt, 1.0)
    h = mean @ wl_ref[...] + bl_ref[...] + x_ref[...] @ wr_ref[...]
    h = jnp.maximum(h, 0.0)
    bcol = b_ref[...]
    parts = []
    for g in range(G):
        hg = jnp.where(bcol == jnp.float32(g), h, 0.0)
        parts.append(jnp.max(hg, axis=0, keepdims=True))
    blockpool = jnp.concatenate(parts, axis=0)
    pooled[...] = jnp.maximum(pooled[...], blockpool)

    @pl.when(i == TC_GRID - 1)
    def _():
        logits = pooled[...] @ w2_ref[...] + b2_ref[...]
        m = jnp.max(logits, axis=-1, keepdims=True)
        lse = jnp.log(jnp.sum(jnp.exp(logits - m), axis=-1, keepdims=True)) + m
        out_ref[...] = logits - lse


def _tc_head(agg2, cnt2d, x_pad, bcol2d, W_l, b_l, W_r, W2, b2):
    return pl.pallas_call(
        _tc_body,
        grid=(TC_GRID,),
        in_specs=[
            pl.BlockSpec((NC, TC_BLK, D), lambda i: (0, i, 0)),
            pl.BlockSpec((TC_BLK, 1), lambda i: (i, 0)),
            pl.BlockSpec((TC_BLK, D), lambda i: (i, 0)),
            pl.BlockSpec((TC_BLK, 1), lambda i: (i, 0)),
            pl.BlockSpec((D, H), lambda i: (0, 0)),
            pl.BlockSpec((1, H), lambda i: (0, 0)),
            pl.BlockSpec((D, H), lambda i: (0, 0)),
            pl.BlockSpec((H, C), lambda i: (0, 0)),
            pl.BlockSpec((1, C), lambda i: (0, 0)),
        ],
        out_specs=pl.BlockSpec((G, C), lambda i: (0, 0)),
        out_shape=jax.ShapeDtypeStruct((G, C), jnp.float32),
        scratch_shapes=[pltpu.VMEM((G, H), jnp.float32)],
    )(agg2, cnt2d, x_pad, bcol2d, W_l, b_l, W_r, W2, b2)


@jax.jit
def kernel(x, edge_index, batch, W_l, b_l, W_r, W2, b2):
    src = edge_index[0].astype(jnp.int32)
    dst = edge_index[1].astype(jnp.int32)
    pad_e = EP - E
    src_p = jnp.concatenate([src, jnp.zeros((pad_e,), jnp.int32)])
    dst_p = jnp.concatenate([dst, jnp.full((pad_e,), N, jnp.int32)])

    agg2 = _sc_aggregate(x, src_p, dst_p)

    cnt_plane = _tc_count(dst.reshape(E, 1))
    cnt2d = cnt_plane.reshape(RP, 1)

    x_pad = jnp.concatenate([x, jnp.zeros((RP - N, D), jnp.float32)])
    bcol = jnp.concatenate([batch.astype(jnp.float32),
                            jnp.full((RP - N,), jnp.float32(G))])
    return _tc_head(agg2, cnt2d, x_pad, bcol[:, None],
                    W_l, b_l.reshape(1, H), W_r, W2, b2.reshape(1, C))


# speedup vs baseline: 3.4130x; 1.0010x over previous
import jax
import jax.numpy as jnp
from jax import lax
from jax.experimental import pallas as pl
from jax.experimental.pallas import tpu as pltpu
from jax.experimental.pallas import tpu_sc as plsc

N = 10000
E = 320000
D = 128
H = 128
C = 2
G = 64

NC = 2
NS = 16
NW = NC * NS
CHUNK = 128
CHUNKS_PER_W = 80
EP = NW * CHUNKS_PER_W * CHUNK
RP = 10240
ROWS_PER_S = RP // NS

TC_BLK = 1024
TC_GRID = RP // TC_BLK

CNT_BLK = 8000
CNT_GRID = E // CNT_BLK
CNT_HI = RP // 128


NBUF = 2


def _sc_body(x_hbm, src_hbm, dst_hbm, z128,
             agg_out, src_v, dst_v, rows_a, rows_b,
             agg_sp, sem_a, sem_b):
    rows = (rows_a, rows_b)
    sems = (sem_a, sem_b)
    c = lax.axis_index("c")
    s = lax.axis_index("s")
    wid = s * NC + c
    base_r = s * ROWS_PER_S

    def zbody(k, carry):
        pltpu.sync_copy(z128, agg_sp.at[pl.ds(base_r + k * 128, 128)])
        return carry

    lax.fori_loop(0, ROWS_PER_S // 128, zbody, 0)
    plsc.subcore_barrier()

    HALF = CHUNKS_PER_W // 2
    for ph in range(2):
        poff = wid * CHUNKS_PER_W + ph * HALF
        pltpu.sync_copy(src_hbm.at[pl.ds(poff, HALF)], src_v)
        pltpu.sync_copy(dst_hbm.at[pl.ds(poff, HALF)], dst_v)
        for b in range(NBUF):
            pltpu.async_copy(x_hbm.at[src_v.at[b]], rows[b], sems[b])

        def pair(j, carry):
            for b in range(NBUF):
                i = j * NBUF + b
                pltpu.make_async_copy(x_hbm.at[src_v.at[i]], rows[b],
                                      sems[b]).wait()
                pltpu.sync_copy(rows[b], agg_sp.at[dst_v.at[i]], add=True)
                nxt = i + NBUF

                @pl.when(nxt < HALF)
                def _():
                    pltpu.async_copy(x_hbm.at[src_v.at[nxt]], rows[b], sems[b])
            return carry

        lax.fori_loop(0, HALF // NBUF, pair, 0)
    plsc.subcore_barrier()

    pltpu.sync_copy(agg_sp.at[pl.ds(base_r, ROWS_PER_S)],
                    agg_out.at[c, pl.ds(base_r, ROWS_PER_S)])


def _sc_aggregate(x, src_p, dst_p):
    mesh = plsc.VectorSubcoreMesh(core_axis_name="c", subcore_axis_name="s")
    z128 = jnp.zeros((128, D), jnp.float32)
    fn = pl.kernel(
        _sc_body,
        out_type=[jax.ShapeDtypeStruct((NC, RP, D), jnp.float32)],
        mesh=mesh,
        scratch_types=[
            pltpu.VMEM((CHUNKS_PER_W // 2, CHUNK), jnp.int32),
            pltpu.VMEM((CHUNKS_PER_W // 2, CHUNK), jnp.int32),
            pltpu.VMEM((CHUNK, D), jnp.float32),
            pltpu.VMEM((CHUNK, D), jnp.float32),
            pltpu.VMEM_SHARED((RP, D), jnp.float32),
            pltpu.SemaphoreType.DMA,
            pltpu.SemaphoreType.DMA,
        ],
    )
    return fn(x, src_p.reshape(EP // CHUNK, CHUNK),
              dst_p.reshape(EP // CHUNK, CHUNK), z128)[0]


def _cnt_body(dst_ref, out_ref, acc):
    i = pl.program_id(0)

    @pl.when(i == 0)
    def _():
        acc[...] = jnp.zeros_like(acc)

    d = dst_ref[...]
    hi = lax.shift_right_logical(d, 7)
    lo = jnp.bitwise_and(d, 127)
    a = (hi == lax.broadcasted_iota(jnp.int32, (CNT_BLK, CNT_HI), 1))
    b = (lo == lax.broadcasted_iota(jnp.int32, (CNT_BLK, 128), 1))
    acc[...] += lax.dot_general(
        a.astype(jnp.bfloat16), b.astype(jnp.bfloat16),
        (((0,), (0,)), ((), ())),
        preferred_element_type=jnp.float32)

    @pl.when(i == CNT_GRID - 1)
    def _():
        out_ref[...] = acc[...]


def _tc_count(dst2d):
    return pl.pallas_call(
        _cnt_body,
        grid=(CNT_GRID,),
        in_specs=[pl.BlockSpec((CNT_BLK, 1), lambda i: (i, 0))],
        out_specs=pl.BlockSpec((CNT_HI, 128), lambda i: (0, 0)),
        out_shape=jax.ShapeDtypeStruct((CNT_HI, 128), jnp.float32),
        scratch_shapes=[pltpu.VMEM((CNT_HI, 128), jnp.float32)],
    )(dst2d)


def _tc_body(agg_ref, cnt_ref, x_ref, b_ref, wl_ref, bl_ref, wr_ref,
             w2_ref, b2_ref, out_ref, pooled):
    i = pl.program_id(0)

    @pl.when(i == 0)
    def _():
        pooled[...] = jnp.zeros_like(pooled)

    a = agg_ref[0] + agg_ref[1]
    cnt = cnt_ref[...]
    mean = a / jnp.maximum(cnt, 1.0)
    h = mean @ wl_ref[...] + bl_ref[...] + x_ref[...] @ wr_ref[...]
    h = jnp.maximum(h, 0.0)
    bcol = b_ref[...]
    parts = []
    for g in range(G):
        hg = jnp.where(bcol == jnp.float32(g), h, 0.0)
        parts.append(jnp.max(hg, axis=0, keepdims=True))
    blockpool = jnp.concatenate(parts, axis=0)
    pooled[...] = jnp.maximum(pooled[...], blockpool)

    @pl.when(i == TC_GRID - 1)
    def _():
        logits = pooled[...] @ w2_ref[...] + b2_ref[...]
        m = jnp.max(logits, axis=-1, keepdims=True)
        lse = jnp.log(jnp.sum(jnp.exp(logits - m), axis=-1, keepdims=True)) + m
        out_ref[...] = logits - lse


def _tc_head(agg2, cnt2d, x_pad, bcol2d, W_l, b_l, W_r, W2, b2):
    return pl.pallas_call(
        _tc_body,
        grid=(TC_GRID,),
        in_specs=[
            pl.BlockSpec((NC, TC_BLK, D), lambda i: (0, i, 0)),
            pl.BlockSpec((TC_BLK, 1), lambda i: (i, 0)),
            pl.BlockSpec((TC_BLK, D), lambda i: (i, 0)),
            pl.BlockSpec((TC_BLK, 1), lambda i: (i, 0)),
            pl.BlockSpec((D, H), lambda i: (0, 0)),
            pl.BlockSpec((1, H), lambda i: (0, 0)),
            pl.BlockSpec((D, H), lambda i: (0, 0)),
            pl.BlockSpec((H, C), lambda i: (0, 0)),
            pl.BlockSpec((1, C), lambda i: (0, 0)),
        ],
        out_specs=pl.BlockSpec((G, C), lambda i: (0, 0)),
        out_shape=jax.ShapeDtypeStruct((G, C), jnp.float32),
        scratch_shapes=[pltpu.VMEM((G, H), jnp.float32)],
    )(agg2, cnt2d, x_pad, bcol2d, W_l, b_l, W_r, W2, b2)


@jax.jit
def kernel(x, edge_index, batch, W_l, b_l, W_r, W2, b2):
    src = edge_index[0].astype(jnp.int32)
    dst = edge_index[1].astype(jnp.int32)
    pad_e = EP - E
    src_p = jnp.concatenate([src, jnp.zeros((pad_e,), jnp.int32)])
    dst_p = jnp.concatenate([dst, jnp.full((pad_e,), N, jnp.int32)])

    cnt_plane = _tc_count(dst.reshape(E, 1))
    cnt2d = cnt_plane.reshape(RP, 1)

    agg2 = _sc_aggregate(x, src_p, dst_p)

    x_pad = jnp.concatenate([x, jnp.zeros((RP - N, D), jnp.float32)])
    bcol = jnp.concatenate([batch.astype(jnp.float32),
                            jnp.full((RP - N,), jnp.float32(G))])
    return _tc_head(agg2, cnt2d, x_pad, bcol[:, None],
                    W_l, b_l.reshape(1, H), W_r, W2, b2.reshape(1, C))
